# baseline (speedup 1.0000x reference)
import functools

import jax
import jax.numpy as jnp
from jax.experimental import pallas as pl
from jax.experimental.pallas import tpu as pltpu

_EPS = 1e-5
_NEG = -1e9


def _ln(x, g, b):
    mu = jnp.mean(x, axis=-1, keepdims=True)
    var = jnp.mean((x - mu) ** 2, axis=-1, keepdims=True)
    return (x - mu) * jax.lax.rsqrt(var + _EPS) * g + b


def _ln_qkv_kernel(x_ref, g_ref, b_ref, wq_ref, wk_ref, wv_ref,
                   q_ref, k_ref, v_ref):
    xn = _ln(x_ref[...], g_ref[...], b_ref[...])
    q_ref[...] = jnp.dot(xn, wq_ref[...], preferred_element_type=jnp.float32)
    k_ref[...] = jnp.dot(xn, wk_ref[...], preferred_element_type=jnp.float32)
    v_ref[...] = jnp.dot(xn, wv_ref[...], preferred_element_type=jnp.float32)


def _ln_qkv(x, g, b, wq, wk, wv, blk=256):
    T, D = x.shape
    shp = jax.ShapeDtypeStruct((T, D), jnp.float32)
    return pl.pallas_call(
        _ln_qkv_kernel,
        grid=(T // blk,),
        in_specs=[
            pl.BlockSpec((blk, D), lambda i: (i, 0)),
            pl.BlockSpec((1, D), lambda i: (0, 0)),
            pl.BlockSpec((1, D), lambda i: (0, 0)),
            pl.BlockSpec((D, D), lambda i: (0, 0)),
            pl.BlockSpec((D, D), lambda i: (0, 0)),
            pl.BlockSpec((D, D), lambda i: (0, 0)),
        ],
        out_specs=[
            pl.BlockSpec((blk, D), lambda i: (i, 0)),
            pl.BlockSpec((blk, D), lambda i: (i, 0)),
            pl.BlockSpec((blk, D), lambda i: (i, 0)),
        ],
        out_shape=[shp, shp, shp],
        compiler_params=pltpu.CompilerParams(
            dimension_semantics=("parallel",)),
    )(x, g[None, :], b[None, :], wq, wk, wv)


def _attn_kernel(q_ref, k_ref, v_ref, wo_ref, r_ref, o_ref, ctx_ref, *,
                 blk_q, kw, row0, n_heads, d_head, scale):
    i = pl.program_id(0)
    row = (jax.lax.broadcasted_iota(jnp.int32, (blk_q, kw), 0)
           + i * blk_q + row0)
    col = jax.lax.broadcasted_iota(jnp.int32, (blk_q, kw), 1)
    mask = col <= row
    for h in range(n_heads):
        sl = slice(h * d_head, (h + 1) * d_head)
        q = q_ref[:, sl] * scale
        s = jax.lax.dot_general(q, k_ref[:, sl], (((1,), (1,)), ((), ())),
                                preferred_element_type=jnp.float32)
        s = jnp.where(mask, s, _NEG)
        m = jnp.max(s, axis=-1, keepdims=True)
        p = jnp.exp(s - m)
        l = jnp.sum(p, axis=-1, keepdims=True)
        ctx_ref[:, sl] = jnp.dot(p, v_ref[:, sl],
                                 preferred_element_type=jnp.float32) / l
    o_ref[...] = (jnp.dot(ctx_ref[...], wo_ref[...],
                          preferred_element_type=jnp.float32) + r_ref[...])


def _attn_call(q, k, v, wo, res, row0, n_heads, d_head, blk_q):
    nq, D = q.shape
    kw = k.shape[0]
    kern = functools.partial(_attn_kernel, blk_q=blk_q, kw=kw, row0=row0,
                             n_heads=n_heads, d_head=d_head,
                             scale=1.0 / (d_head ** 0.5))
    return pl.pallas_call(
        kern,
        grid=(nq // blk_q,),
        in_specs=[
            pl.BlockSpec((blk_q, D), lambda i: (i, 0)),
            pl.BlockSpec((kw, D), lambda i: (0, 0)),
            pl.BlockSpec((kw, D), lambda i: (0, 0)),
            pl.BlockSpec((D, D), lambda i: (0, 0)),
            pl.BlockSpec((blk_q, D), lambda i: (i, 0)),
        ],
        out_specs=pl.BlockSpec((blk_q, D), lambda i: (i, 0)),
        out_shape=jax.ShapeDtypeStruct((nq, D), jnp.float32),
        scratch_shapes=[pltpu.VMEM((blk_q, D), jnp.float32)],
        compiler_params=pltpu.CompilerParams(
            dimension_semantics=("parallel",)),
    )(q, k, v, wo, res)


def _attention(q, k, v, wo, res, n_heads, d_head, blk_q=256):
    T, D = q.shape
    half = T // 2
    lo = _attn_call(q[:half], k[:half], v[:half], wo, res[:half], 0,
                    n_heads, d_head, blk_q)
    hi = _attn_call(q[half:], k, v, wo, res[half:], half,
                    n_heads, d_head, blk_q)
    return lo, hi


def _ln_router_kernel(xlo_ref, xhi_ref, g_ref, b_ref, wr_ref,
                      xln_ref, eidx_ref, posc_ref, keep_ref, gatek_ref,
                      aux_ref, *, n_exp, cap, lanes, chunk):
    x = jnp.concatenate([xlo_ref[...], xhi_ref[...]], axis=0)
    T = x.shape[0]
    xn = _ln(x, g_ref[...], b_ref[...])
    xln_ref[...] = xn
    logits = jnp.dot(xn, wr_ref[...], preferred_element_type=jnp.float32)
    lane = jax.lax.broadcasted_iota(jnp.int32, (T, lanes), 1)
    logits = jnp.where(lane < n_exp, logits, -1e30)
    m = jnp.max(logits, axis=-1, keepdims=True)
    ex = jnp.exp(logits - m)
    probs = ex / jnp.sum(ex, axis=-1, keepdims=True)
    gate = jnp.max(probs, axis=-1, keepdims=True)
    eidx = jnp.min(jnp.where(probs == gate, lane, lanes), axis=-1,
                   keepdims=True)
    onehot = (lane == eidx).astype(jnp.float32)

    fm = jnp.mean(onehot, axis=0, keepdims=True)
    pm = jnp.mean(probs, axis=0, keepdims=True)
    aux = jnp.float32(n_exp) * jnp.sum(fm * pm)
    aux_ref[...] = jnp.full((1, lanes), aux, jnp.float32)

    r = jax.lax.broadcasted_iota(jnp.int32, (chunk, chunk), 0)
    c = jax.lax.broadcasted_iota(jnp.int32, (chunk, chunk), 1)
    tri = (c <= r).astype(jnp.float32)
    carry = jnp.zeros((1, lanes), jnp.float32)
    for i in range(T // chunk):
        sl = slice(i * chunk, (i + 1) * chunk)
        oh = onehot[sl]
        cs = jnp.dot(tri, oh, preferred_element_type=jnp.float32) + carry
        pos = (jnp.sum(cs * oh, axis=-1, keepdims=True) - 1.0
               ).astype(jnp.int32)
        keep = pos < cap
        posc_ref[sl, :] = jnp.where(keep, pos, 0)
        keepf = keep.astype(jnp.float32)
        keep_ref[sl, :] = keepf
        eidx_ref[sl, :] = eidx[sl]
        gatek_ref[sl, :] = gate[sl] * keepf
        carry = carry + jnp.sum(oh, axis=0, keepdims=True)


def _ln_router(xlo, xhi, g, b, wr_pad, n_exp, cap):
    T = xlo.shape[0] * 2
    D = xlo.shape[1]
    lanes = wr_pad.shape[1]
    kern = functools.partial(_ln_router_kernel, n_exp=n_exp, cap=cap,
                             lanes=lanes, chunk=256)
    return pl.pallas_call(
        kern,
        out_shape=[
            jax.ShapeDtypeStruct((T, D), jnp.float32),
            jax.ShapeDtypeStruct((T, 1), jnp.int32),
            jax.ShapeDtypeStruct((T, 1), jnp.int32),
            jax.ShapeDtypeStruct((T, 1), jnp.float32),
            jax.ShapeDtypeStruct((T, 1), jnp.float32),
            jax.ShapeDtypeStruct((1, lanes), jnp.float32),
        ],
    )(xlo, xhi, g[None, :], b[None, :], wr_pad)


def _moe_kernel(xln_ref, er_ref, pr_ref, kr_ref, ec_ref, pc_ref, gc_ref,
                w1_ref, b1_ref, w2_ref, b2_ref, rlo_ref, rhi_ref, o_ref,
                oe_ref, *, cap, n_f):
    e = pl.program_id(0)
    f = pl.program_id(1)
    T = xln_ref.shape[0]
    crow = jax.lax.broadcasted_iota(jnp.int32, (cap, T), 0)
    d = jnp.where((er_ref[...] == e) & (pr_ref[...] == crow),
                  kr_ref[...], 0.0)
    xe = jnp.dot(d, xln_ref[...], preferred_element_type=jnp.float32)
    h = jnp.maximum(
        jnp.dot(xe, w1_ref[0], preferred_element_type=jnp.float32)
        + b1_ref[0], 0.0)
    part = jnp.dot(h, w2_ref[0], preferred_element_type=jnp.float32)

    @pl.when(f == 0)
    def _():
        oe_ref[...] = part + b2_ref[0]

    @pl.when(f != 0)
    def _():
        oe_ref[...] = oe_ref[...] + part

    @pl.when(f == n_f - 1)
    def _():
        clane = jax.lax.broadcasted_iota(jnp.int32, (T, cap), 1)
        gmat = jnp.where((ec_ref[...] == e) & (pc_ref[...] == clane),
                         gc_ref[...], 0.0)
        y = jnp.dot(gmat, oe_ref[...], preferred_element_type=jnp.float32)

        @pl.when(e == 0)
        def _():
            o_ref[...] = jnp.concatenate(
                [rlo_ref[...], rhi_ref[...]], axis=0) + y

        @pl.when(e != 0)
        def _():
            o_ref[...] = o_ref[...] + y


def _moe(xln, er, pr, kr, ec, pc, gc, w1, b1, w2, b2, rlo, rhi, cap, n_f=2):
    T, D = xln.shape
    E, _, F = w1.shape
    fb = F // n_f
    kern = functools.partial(_moe_kernel, cap=cap, n_f=n_f)
    return pl.pallas_call(
        kern,
        grid=(E, n_f),
        in_specs=[
            pl.BlockSpec((T, D), lambda e, f: (0, 0)),
            pl.BlockSpec((1, T), lambda e, f: (0, 0)),
            pl.BlockSpec((1, T), lambda e, f: (0, 0)),
            pl.BlockSpec((1, T), lambda e, f: (0, 0)),
            pl.BlockSpec((T, 1), lambda e, f: (0, 0)),
            pl.BlockSpec((T, 1), lambda e, f: (0, 0)),
            pl.BlockSpec((T, 1), lambda e, f: (0, 0)),
            pl.BlockSpec((1, D, fb), lambda e, f: (e, 0, f)),
            pl.BlockSpec((1, 1, fb), lambda e, f: (e, 0, f)),
            pl.BlockSpec((1, fb, D), lambda e, f: (e, f, 0)),
            pl.BlockSpec((1, 1, D), lambda e, f: (e, 0, 0)),
            pl.BlockSpec((T // 2, D), lambda e, f: (0, 0)),
            pl.BlockSpec((T // 2, D), lambda e, f: (0, 0)),
        ],
        out_specs=pl.BlockSpec((T, D), lambda e, f: (0, 0)),
        out_shape=jax.ShapeDtypeStruct((T, D), jnp.float32),
        scratch_shapes=[pltpu.VMEM((cap, D), jnp.float32)],
        compiler_params=pltpu.CompilerParams(
            dimension_semantics=("arbitrary", "arbitrary")),
    )(xln, er, pr, kr, ec, pc, gc, w1, b1.reshape(E, 1, F), w2,
      b2.reshape(E, 1, D), rlo, rhi)


def _decoder_layer(x, p, n_heads, d_head, n_exp, cap_factor):
    T, D = x.shape
    cap = int(cap_factor * T / n_exp)
    q, k, v = _ln_qkv(x, p['ln1_g'], p['ln1_b'], p['Wq'], p['Wk'], p['Wv'])
    x2lo, x2hi = _attention(q, k, v, p['Wo'], x, n_heads, d_head)
    lanes = 128
    wr_pad = jnp.pad(p['Wr'], ((0, 0), (0, lanes - n_exp)))
    xln, eidx, posc, keepf, gatek, auxrow = _ln_router(
        x2lo, x2hi, p['ln2_g'], p['ln2_b'], wr_pad, n_exp, cap)
    out = _moe(xln, eidx.T, posc.T, keepf.T, eidx, posc, gatek,
               p['W1'], p['b1'], p['W2'], p['b2'], x2lo, x2hi, cap)
    return out, auxrow[0, 0]


def kernel(hidden_states, params):
    B, S, D = hidden_states.shape
    T = B * S
    x = hidden_states.reshape(T, D)
    n_exp = params['layers'][0]['W1'].shape[0]
    n_heads = 16
    d_head = D // n_heads
    aux_total = jnp.zeros((), jnp.float32)
    for p in params['layers']:
        x, aux = _decoder_layer(x, p, n_heads, d_head, n_exp, 1.25)
        aux_total = aux_total + aux
    return x.reshape(B, S, D), aux_total


# speedup vs baseline: 1.0130x; 1.0130x over previous
import functools

import jax
import jax.numpy as jnp
from jax.experimental import pallas as pl
from jax.experimental.pallas import tpu as pltpu

_EPS = 1e-5
_NEG = -1e9


def _ln(x, g, b):
    mu = jnp.mean(x, axis=-1, keepdims=True)
    var = jnp.mean((x - mu) ** 2, axis=-1, keepdims=True)
    return (x - mu) * jax.lax.rsqrt(var + _EPS) * g + b


def _ln_qkv_kernel(x_ref, g_ref, b_ref, wq_ref, wk_ref, wv_ref,
                   q_ref, k_ref, v_ref):
    xn = _ln(x_ref[...], g_ref[...], b_ref[...])
    q_ref[...] = jnp.dot(xn, wq_ref[...], preferred_element_type=jnp.float32)
    k_ref[...] = jnp.dot(xn, wk_ref[...], preferred_element_type=jnp.float32)
    v_ref[...] = jnp.dot(xn, wv_ref[...], preferred_element_type=jnp.float32)


def _ln_qkv(x, g, b, wq, wk, wv, blk=256):
    T, D = x.shape
    shp = jax.ShapeDtypeStruct((T, D), jnp.float32)
    return pl.pallas_call(
        _ln_qkv_kernel,
        grid=(T // blk,),
        in_specs=[
            pl.BlockSpec((blk, D), lambda i: (i, 0)),
            pl.BlockSpec((1, D), lambda i: (0, 0)),
            pl.BlockSpec((1, D), lambda i: (0, 0)),
            pl.BlockSpec((D, D), lambda i: (0, 0)),
            pl.BlockSpec((D, D), lambda i: (0, 0)),
            pl.BlockSpec((D, D), lambda i: (0, 0)),
        ],
        out_specs=[
            pl.BlockSpec((blk, D), lambda i: (i, 0)),
            pl.BlockSpec((blk, D), lambda i: (i, 0)),
            pl.BlockSpec((blk, D), lambda i: (i, 0)),
        ],
        out_shape=[shp, shp, shp],
        compiler_params=pltpu.CompilerParams(
            dimension_semantics=("parallel",)),
    )(x, g[None, :], b[None, :], wq, wk, wv)


def _attn_kernel(q_ref, k_ref, v_ref, o_ref, *, blk_q, kw, row0, n_heads,
                 d_head, scale):
    i = pl.program_id(0)
    row = (jax.lax.broadcasted_iota(jnp.int32, (blk_q, kw), 0)
           + i * blk_q + row0)
    col = jax.lax.broadcasted_iota(jnp.int32, (blk_q, kw), 1)
    mask = col <= row
    for h in range(n_heads):
        sl = slice(h * d_head, (h + 1) * d_head)
        q = q_ref[:, sl] * scale
        s = jax.lax.dot_general(q, k_ref[:, sl], (((1,), (1,)), ((), ())),
                                preferred_element_type=jnp.float32)
        s = jnp.where(mask, s, _NEG)
        m = jnp.max(s, axis=-1, keepdims=True)
        p = jnp.exp(s - m)
        l = jnp.sum(p, axis=-1, keepdims=True)
        o_ref[:, sl] = jnp.dot(p, v_ref[:, sl],
                               preferred_element_type=jnp.float32) / l


def _attn_call(q, k, v, row0, n_heads, d_head, blk_q):
    nq, D = q.shape
    kw = k.shape[0]
    kern = functools.partial(_attn_kernel, blk_q=blk_q, kw=kw, row0=row0,
                             n_heads=n_heads, d_head=d_head,
                             scale=1.0 / (d_head ** 0.5))
    return pl.pallas_call(
        kern,
        grid=(nq // blk_q,),
        in_specs=[
            pl.BlockSpec((blk_q, D), lambda i: (i, 0)),
            pl.BlockSpec((kw, D), lambda i: (0, 0)),
            pl.BlockSpec((kw, D), lambda i: (0, 0)),
        ],
        out_specs=pl.BlockSpec((blk_q, D), lambda i: (i, 0)),
        out_shape=jax.ShapeDtypeStruct((nq, D), jnp.float32),
        compiler_params=pltpu.CompilerParams(
            dimension_semantics=("parallel",)),
    )(q, k, v)


def _attention(q, k, v, n_heads, d_head, blk_q=256):
    T, D = q.shape
    half = T // 2
    lo = _attn_call(q[:half], k[:half], v[:half], 0, n_heads, d_head, blk_q)
    hi = _attn_call(q[half:], k, v, half, n_heads, d_head, blk_q)
    return jnp.concatenate([lo, hi], axis=0)


def _proj_add_kernel(a_ref, w_ref, r_ref, o_ref):
    o_ref[...] = (jnp.dot(a_ref[...], w_ref[...],
                          preferred_element_type=jnp.float32) + r_ref[...])


def _proj_add(a, w, r, blk=256):
    T, D = a.shape
    return pl.pallas_call(
        _proj_add_kernel,
        grid=(T // blk,),
        in_specs=[
            pl.BlockSpec((blk, D), lambda i: (i, 0)),
            pl.BlockSpec((D, D), lambda i: (0, 0)),
            pl.BlockSpec((blk, D), lambda i: (i, 0)),
        ],
        out_specs=pl.BlockSpec((blk, D), lambda i: (i, 0)),
        out_shape=jax.ShapeDtypeStruct((T, D), jnp.float32),
        compiler_params=pltpu.CompilerParams(
            dimension_semantics=("parallel",)),
    )(a, w, r)


def _ln_router_kernel(x_ref, g_ref, b_ref, wr_ref,
                      xln_ref, eidx_ref, posc_ref, keep_ref, gatek_ref,
                      aux_ref, *, n_exp, cap, lanes, chunk):
    x = x_ref[...]
    T = x.shape[0]
    xn = _ln(x, g_ref[...], b_ref[...])
    xln_ref[...] = xn
    logits = jnp.dot(xn, wr_ref[...], preferred_element_type=jnp.float32)
    lane = jax.lax.broadcasted_iota(jnp.int32, (T, lanes), 1)
    logits = jnp.where(lane < n_exp, logits, -1e30)
    m = jnp.max(logits, axis=-1, keepdims=True)
    ex = jnp.exp(logits - m)
    probs = ex / jnp.sum(ex, axis=-1, keepdims=True)
    gate = jnp.max(probs, axis=-1, keepdims=True)
    eidx = jnp.min(jnp.where(probs == gate, lane, lanes), axis=-1,
                   keepdims=True)
    onehot = (lane == eidx).astype(jnp.float32)

    fm = jnp.mean(onehot, axis=0, keepdims=True)
    pm = jnp.mean(probs, axis=0, keepdims=True)
    aux = jnp.float32(n_exp) * jnp.sum(fm * pm)
    aux_ref[...] = jnp.full((1, lanes), aux, jnp.float32)

    r = jax.lax.broadcasted_iota(jnp.int32, (chunk, chunk), 0)
    c = jax.lax.broadcasted_iota(jnp.int32, (chunk, chunk), 1)
    tri = (c <= r).astype(jnp.float32)
    carry = jnp.zeros((1, lanes), jnp.float32)
    for i in range(T // chunk):
        sl = slice(i * chunk, (i + 1) * chunk)
        oh = onehot[sl]
        cs = jnp.dot(tri, oh, preferred_element_type=jnp.float32) + carry
        pos = (jnp.sum(cs * oh, axis=-1, keepdims=True) - 1.0
               ).astype(jnp.int32)
        keep = pos < cap
        posc_ref[sl, :] = jnp.where(keep, pos, 0)
        keepf = keep.astype(jnp.float32)
        keep_ref[sl, :] = keepf
        eidx_ref[sl, :] = eidx[sl]
        gatek_ref[sl, :] = gate[sl] * keepf
        carry = carry + jnp.sum(oh, axis=0, keepdims=True)


def _ln_router(x, g, b, wr_pad, n_exp, cap):
    T, D = x.shape
    lanes = wr_pad.shape[1]
    kern = functools.partial(_ln_router_kernel, n_exp=n_exp, cap=cap,
                             lanes=lanes, chunk=256)
    return pl.pallas_call(
        kern,
        out_shape=[
            jax.ShapeDtypeStruct((T, D), jnp.float32),
            jax.ShapeDtypeStruct((T, 1), jnp.int32),
            jax.ShapeDtypeStruct((T, 1), jnp.int32),
            jax.ShapeDtypeStruct((T, 1), jnp.float32),
            jax.ShapeDtypeStruct((T, 1), jnp.float32),
            jax.ShapeDtypeStruct((1, lanes), jnp.float32),
        ],
    )(x, g[None, :], b[None, :], wr_pad)


def _moe_kernel(xln_ref, er_ref, pr_ref, kr_ref, ec_ref, pc_ref, gc_ref,
                w1_ref, b1_ref, w2_ref, b2_ref, res_ref, o_ref,
                oe_ref, *, cap, n_f):
    e = pl.program_id(0)
    f = pl.program_id(1)
    T = xln_ref.shape[0]
    crow = jax.lax.broadcasted_iota(jnp.int32, (cap, T), 0)
    d = jnp.where((er_ref[...] == e) & (pr_ref[...] == crow),
                  kr_ref[...], 0.0)
    xe = jnp.dot(d, xln_ref[...], preferred_element_type=jnp.float32)
    h = jnp.maximum(
        jnp.dot(xe, w1_ref[0], preferred_element_type=jnp.float32)
        + b1_ref[0], 0.0)
    part = jnp.dot(h, w2_ref[0], preferred_element_type=jnp.float32)

    @pl.when(f == 0)
    def _():
        oe_ref[...] = part + b2_ref[0]

    @pl.when(f != 0)
    def _():
        oe_ref[...] = oe_ref[...] + part

    @pl.when(f == n_f - 1)
    def _():
        clane = jax.lax.broadcasted_iota(jnp.int32, (T, cap), 1)
        gmat = jnp.where((ec_ref[...] == e) & (pc_ref[...] == clane),
                         gc_ref[...], 0.0)
        y = jnp.dot(gmat, oe_ref[...], preferred_element_type=jnp.float32)

        @pl.when(e == 0)
        def _():
            o_ref[...] = res_ref[...] + y

        @pl.when(e != 0)
        def _():
            o_ref[...] = o_ref[...] + y


def _moe(xln, er, pr, kr, ec, pc, gc, w1, b1, w2, b2, res, cap, n_f=2):
    T, D = xln.shape
    E, _, F = w1.shape
    fb = F // n_f
    kern = functools.partial(_moe_kernel, cap=cap, n_f=n_f)
    return pl.pallas_call(
        kern,
        grid=(E, n_f),
        in_specs=[
            pl.BlockSpec((T, D), lambda e, f: (0, 0)),
            pl.BlockSpec((1, T), lambda e, f: (0, 0)),
            pl.BlockSpec((1, T), lambda e, f: (0, 0)),
            pl.BlockSpec((1, T), lambda e, f: (0, 0)),
            pl.BlockSpec((T, 1), lambda e, f: (0, 0)),
            pl.BlockSpec((T, 1), lambda e, f: (0, 0)),
            pl.BlockSpec((T, 1), lambda e, f: (0, 0)),
            pl.BlockSpec((1, D, fb), lambda e, f: (e, 0, f)),
            pl.BlockSpec((1, 1, fb), lambda e, f: (e, 0, f)),
            pl.BlockSpec((1, fb, D), lambda e, f: (e, f, 0)),
            pl.BlockSpec((1, 1, D), lambda e, f: (e, 0, 0)),
            pl.BlockSpec((T, D), lambda e, f: (0, 0)),
        ],
        out_specs=pl.BlockSpec((T, D), lambda e, f: (0, 0)),
        out_shape=jax.ShapeDtypeStruct((T, D), jnp.float32),
        scratch_shapes=[pltpu.VMEM((cap, D), jnp.float32)],
        compiler_params=pltpu.CompilerParams(
            dimension_semantics=("arbitrary", "arbitrary")),
    )(xln, er, pr, kr, ec, pc, gc, w1, b1.reshape(E, 1, F), w2,
      b2.reshape(E, 1, D), res)


def _decoder_layer(x, p, n_heads, d_head, n_exp, cap_factor):
    T, D = x.shape
    cap = int(cap_factor * T / n_exp)
    q, k, v = _ln_qkv(x, p['ln1_g'], p['ln1_b'], p['Wq'], p['Wk'], p['Wv'])
    attn = _attention(q, k, v, n_heads, d_head)
    x2 = _proj_add(attn, p['Wo'], x)
    lanes = 128
    wr_pad = jnp.pad(p['Wr'], ((0, 0), (0, lanes - n_exp)))
    xln, eidx, posc, keepf, gatek, auxrow = _ln_router(
        x2, p['ln2_g'], p['ln2_b'], wr_pad, n_exp, cap)
    out = _moe(xln, eidx.T, posc.T, keepf.T, eidx, posc, gatek,
               p['W1'], p['b1'], p['W2'], p['b2'], x2, cap)
    return out, auxrow[0, 0]


def kernel(hidden_states, params):
    B, S, D = hidden_states.shape
    T = B * S
    x = hidden_states.reshape(T, D)
    n_exp = params['layers'][0]['W1'].shape[0]
    n_heads = 16
    d_head = D // n_heads
    aux_total = jnp.zeros((), jnp.float32)
    for p in params['layers']:
        x, aux = _decoder_layer(x, p, n_heads, d_head, n_exp, 1.25)
        aux_total = aux_total + aux
    return x.reshape(B, S, D), aux_total
